# Initial kernel scaffold; baseline (speedup 1.0000x reference)
import jax
import jax.numpy as jnp
from jax.experimental import pallas as pl


def kernel(feat_x, feat_y):
    raise NotImplementedError("write your pallas kernel here")


# speedup vs baseline: 1.2073x; 1.2073x over previous
import jax
import jax.numpy as jnp
from jax.experimental import pallas as pl

_TAU = 0.2
_K = 15
_NX = 1024
_NY = 100000
_C = 16
_LANES = 128
_NYP = ((_NY + _LANES - 1) // _LANES) * _LANES
_BX = 8


def _body(x_ref, yt_ref, bias_ref, val_ref, idx_ref):
    x = x_ref[...]
    yt = yt_ref[...]
    sim = jax.lax.dot_general(
        x, yt, (((1,), (0,)), ((), ())),
        preferred_element_type=jnp.float32,
    ) / _TAU
    sim = sim + bias_ref[...]
    colidx = jax.lax.broadcasted_iota(jnp.int32, (_BX, _NYP), 1)

    vals = []
    idxs = []
    for _ in range(_K):
        m = jnp.max(sim, axis=1, keepdims=True)
        cand = jnp.where(sim == m, colidx, _NYP)
        am = jnp.min(cand, axis=1, keepdims=True)
        vals.append(m)
        idxs.append(am)
        sim = jnp.where(colidx == am, -jnp.inf, sim)

    v = jnp.concatenate(vals, axis=1)
    i = jnp.concatenate(idxs, axis=1)
    e = jnp.exp(v - v[:, :1])
    sm = e / jnp.sum(e, axis=1, keepdims=True)

    val_ref[...] = jnp.concatenate(
        [sm, jnp.zeros((_BX, _LANES - _K), jnp.float32)], axis=1)
    idx_ref[...] = jnp.concatenate(
        [i, jnp.zeros((_BX, _LANES - _K), jnp.int32)], axis=1)


def kernel(feat_x, feat_y):
    yt = jnp.pad(feat_y.T, ((0, 0), (0, _NYP - _NY)))
    bias = jnp.where(jnp.arange(_NYP) < _NY, 0.0, -1e30)[None, :]
    bias = bias.astype(jnp.float32)

    grid = (_NX // _BX,)
    sm_pad, idx_pad = pl.pallas_call(
        _body,
        grid=grid,
        in_specs=[
            pl.BlockSpec((_BX, _C), lambda i: (i, 0)),
            pl.BlockSpec((_C, _NYP), lambda i: (0, 0)),
            pl.BlockSpec((1, _NYP), lambda i: (0, 0)),
        ],
        out_specs=[
            pl.BlockSpec((_BX, _LANES), lambda i: (i, 0)),
            pl.BlockSpec((_BX, _LANES), lambda i: (i, 0)),
        ],
        out_shape=[
            jax.ShapeDtypeStruct((_NX, _LANES), jnp.float32),
            jax.ShapeDtypeStruct((_NX, _LANES), jnp.int32),
        ],
    )(feat_x, yt, bias)

    row_idx = jnp.broadcast_to(
        jnp.arange(_NX, dtype=jnp.int64)[:, None], (_NX, _K))
    return (row_idx,
            idx_pad[:, :_K].astype(jnp.int64),
            sm_pad[:, :_K])


# speedup vs baseline: 1.2297x; 1.0185x over previous
import jax
import jax.numpy as jnp
from jax.experimental import pallas as pl
from jax.experimental.pallas import tpu as pltpu

_TAU = 0.2
_K = 15
_NX = 1024
_NY = 100000
_C = 16
_LANES = 128
_NYP = ((_NY + _LANES - 1) // _LANES) * _LANES
_BX = 8
_S = 512
_NCH = _NYP // _S


def _softmax_desc(v):
    e = jnp.exp(v - v[:, :1])
    return e / jnp.sum(e, axis=1, keepdims=True)


def _write_out(val_ref, idx_ref, v, i):
    val_ref[...] = jnp.concatenate(
        [_softmax_desc(v), jnp.zeros((_BX, _LANES - _K), jnp.float32)], axis=1)
    idx_ref[...] = jnp.concatenate(
        [i, jnp.zeros((_BX, _LANES - _K), jnp.int32)], axis=1)


def _body(x_ref, yt_ref, bias_ref, val_ref, idx_ref, sim_scr):
    x = x_ref[...]
    yt = yt_ref[...]
    sim = jax.lax.dot_general(
        x, yt, (((1,), (0,)), ((), ())),
        preferred_element_type=jnp.float32,
    ) / _TAU
    sim_scr[...] = sim + bias_ref[...]

    ninf = jnp.float32(-jnp.inf)

    def build_step(j, carry):
        m1, m2, a1, a2 = carry
        v = sim_scr[:, pl.ds(j * _S, _S)]
        jb = jnp.full((_BX, _S), j, jnp.int32)
        gt1 = v > m1
        gt2 = v > m2
        m2n = jnp.where(gt1, m1, jnp.where(gt2, v, m2))
        a2n = jnp.where(gt1, a1, jnp.where(gt2, jb, a2))
        m1n = jnp.where(gt1, v, m1)
        a1n = jnp.where(gt1, jb, a1)
        return (m1n, m2n, a1n, a2n)

    init = (jnp.full((_BX, _S), ninf), jnp.full((_BX, _S), ninf),
            jnp.zeros((_BX, _S), jnp.int32), jnp.zeros((_BX, _S), jnp.int32))
    m1, m2, a1, a2 = jax.lax.fori_loop(0, _NCH, build_step, init)

    seg = jax.lax.broadcasted_iota(jnp.int32, (_BX, _S), 1)
    vals = []
    idxs = []
    for _ in range(_K):
        m = jnp.max(m1, axis=1, keepdims=True)
        s = jnp.min(jnp.where(m1 == m, seg, _S), axis=1, keepdims=True)
        oneh = seg == s
        c = jnp.sum(jnp.where(oneh, a1, 0), axis=1, keepdims=True)
        vals.append(m)
        idxs.append(c * _S + s)
        m1 = jnp.where(oneh, m2, m1)
        a1 = jnp.where(oneh, a2, a1)
        m2 = jnp.where(oneh, ninf, m2)

    v = jnp.concatenate(vals, axis=1)
    i = jnp.concatenate(idxs, axis=1)
    _write_out(val_ref, idx_ref, v, i)

    v15 = vals[-1]

    def count_step(j, cnt):
        chunk = sim_scr[:, pl.ds(j * _S, _S)]
        return cnt + jnp.sum((chunk > v15).astype(jnp.int32),
                             axis=1, keepdims=True)

    cnt = jax.lax.fori_loop(0, _NCH, count_step,
                            jnp.zeros((_BX, 1), jnp.int32))
    redo = jnp.any(cnt > _K - 1)

    @pl.when(redo)
    def _redo():
        simv = sim_scr[...]
        colidx = jax.lax.broadcasted_iota(jnp.int32, (_BX, _NYP), 1)
        rvals = []
        ridxs = []
        sm = simv
        for _ in range(_K):
            mm = jnp.max(sm, axis=1, keepdims=True)
            am = jnp.min(jnp.where(sm == mm, colidx, _NYP),
                         axis=1, keepdims=True)
            rvals.append(mm)
            ridxs.append(am)
            sm = jnp.where(colidx == am, ninf, sm)
        _write_out(val_ref, idx_ref,
                   jnp.concatenate(rvals, axis=1),
                   jnp.concatenate(ridxs, axis=1))


def kernel(feat_x, feat_y):
    yt = jnp.pad(feat_y.T, ((0, 0), (0, _NYP - _NY)))
    bias = jnp.where(jnp.arange(_NYP) < _NY, 0.0, -1e30)[None, :]
    bias = bias.astype(jnp.float32)

    sm_pad, idx_pad = pl.pallas_call(
        _body,
        grid=(_NX // _BX,),
        in_specs=[
            pl.BlockSpec((_BX, _C), lambda i: (i, 0)),
            pl.BlockSpec((_C, _NYP), lambda i: (0, 0)),
            pl.BlockSpec((1, _NYP), lambda i: (0, 0)),
        ],
        out_specs=[
            pl.BlockSpec((_BX, _LANES), lambda i: (i, 0)),
            pl.BlockSpec((_BX, _LANES), lambda i: (i, 0)),
        ],
        out_shape=[
            jax.ShapeDtypeStruct((_NX, _LANES), jnp.float32),
            jax.ShapeDtypeStruct((_NX, _LANES), jnp.int32),
        ],
        scratch_shapes=[pltpu.VMEM((_BX, _NYP), jnp.float32)],
    )(feat_x, yt, bias)

    row_idx = jnp.broadcast_to(
        jnp.arange(_NX, dtype=jnp.int64)[:, None], (_NX, _K))
    return (row_idx,
            idx_pad[:, :_K].astype(jnp.int64),
            sm_pad[:, :_K])


# speedup vs baseline: 2.3137x; 1.8816x over previous
import jax
import jax.numpy as jnp
from jax.experimental import pallas as pl

_TAU = 0.2
_K = 15
_NX = 1024
_NY = 100000
_C = 16
_LANES = 128
_NYP = ((_NY + _LANES - 1) // _LANES) * _LANES
_BX = 8
_S = 512
_NCH = _NYP // _S


def _softmax_desc(v):
    e = jnp.exp(v - v[:, :1])
    return e / jnp.sum(e, axis=1, keepdims=True)


def _write_out(val_ref, idx_ref, v, i):
    val_ref[...] = jnp.concatenate(
        [_softmax_desc(v), jnp.zeros((_BX, _LANES - _K), jnp.float32)], axis=1)
    idx_ref[...] = jnp.concatenate(
        [i, jnp.zeros((_BX, _LANES - _K), jnp.int32)], axis=1)


def _compute_sim(x_ref, yt_ref, bias_ref):
    sim = jax.lax.dot_general(
        x_ref[...], yt_ref[...], (((1,), (0,)), ((), ())),
        preferred_element_type=jnp.float32,
    ) / _TAU
    return sim + bias_ref[...]


def _body(x_ref, yt_ref, bias_ref, val_ref, idx_ref):
    sim = _compute_sim(x_ref, yt_ref, bias_ref)

    ninf = jnp.float32(-jnp.inf)
    m1 = jnp.full((_BX, _S), ninf)
    m2 = jnp.full((_BX, _S), ninf)
    a1 = jnp.zeros((_BX, _S), jnp.int32)
    a2 = jnp.zeros((_BX, _S), jnp.int32)
    for j in range(_NCH):
        v = sim[:, j * _S:(j + 1) * _S]
        jb = jnp.full((_BX, _S), j, jnp.int32)
        gt1 = v > m1
        gt2 = v > m2
        m2 = jnp.where(gt1, m1, jnp.where(gt2, v, m2))
        a2 = jnp.where(gt1, a1, jnp.where(gt2, jb, a2))
        m1 = jnp.where(gt1, v, m1)
        a1 = jnp.where(gt1, jb, a1)

    seg = jax.lax.broadcasted_iota(jnp.int32, (_BX, _S), 1)
    vals = []
    idxs = []
    for _ in range(_K):
        m = jnp.max(m1, axis=1, keepdims=True)
        s = jnp.min(jnp.where(m1 == m, seg, _S), axis=1, keepdims=True)
        oneh = seg == s
        c = jnp.sum(jnp.where(oneh, a1, 0), axis=1, keepdims=True)
        vals.append(m)
        idxs.append(c * _S + s)
        m1 = jnp.where(oneh, m2, m1)
        a1 = jnp.where(oneh, a2, a1)
        m2 = jnp.where(oneh, ninf, m2)

    _write_out(val_ref, idx_ref,
               jnp.concatenate(vals, axis=1),
               jnp.concatenate(idxs, axis=1))

    v15 = vals[-1]
    cnt = jnp.sum((sim > v15).astype(jnp.int32), axis=1, keepdims=True)
    redo = jnp.any(cnt > _K - 1)

    @pl.when(redo)
    def _redo():
        sm = _compute_sim(x_ref, yt_ref, bias_ref)
        colidx = jax.lax.broadcasted_iota(jnp.int32, (_BX, _NYP), 1)
        rvals = []
        ridxs = []
        for _ in range(_K):
            mm = jnp.max(sm, axis=1, keepdims=True)
            am = jnp.min(jnp.where(sm == mm, colidx, _NYP),
                         axis=1, keepdims=True)
            rvals.append(mm)
            ridxs.append(am)
            sm = jnp.where(colidx == am, ninf, sm)
        _write_out(val_ref, idx_ref,
                   jnp.concatenate(rvals, axis=1),
                   jnp.concatenate(ridxs, axis=1))


def kernel(feat_x, feat_y):
    yt = jnp.pad(feat_y.T, ((0, 0), (0, _NYP - _NY)))
    bias = jnp.where(jnp.arange(_NYP) < _NY, 0.0, -1e30)[None, :]
    bias = bias.astype(jnp.float32)

    sm_pad, idx_pad = pl.pallas_call(
        _body,
        grid=(_NX // _BX,),
        in_specs=[
            pl.BlockSpec((_BX, _C), lambda i: (i, 0)),
            pl.BlockSpec((_C, _NYP), lambda i: (0, 0)),
            pl.BlockSpec((1, _NYP), lambda i: (0, 0)),
        ],
        out_specs=[
            pl.BlockSpec((_BX, _LANES), lambda i: (i, 0)),
            pl.BlockSpec((_BX, _LANES), lambda i: (i, 0)),
        ],
        out_shape=[
            jax.ShapeDtypeStruct((_NX, _LANES), jnp.float32),
            jax.ShapeDtypeStruct((_NX, _LANES), jnp.int32),
        ],
    )(feat_x, yt, bias)

    row_idx = jnp.broadcast_to(
        jnp.arange(_NX, dtype=jnp.int64)[:, None], (_NX, _K))
    return (row_idx,
            idx_pad[:, :_K].astype(jnp.int64),
            sm_pad[:, :_K])


# speedup vs baseline: 4.9605x; 2.1440x over previous
import jax
import jax.numpy as jnp
from jax.experimental import pallas as pl

_TAU = 0.2
_K = 15
_NX = 1024
_NY = 100000
_C = 16
_LANES = 128
_NYP = 100352
_BX = 8
_S = 512
_NCH = _NYP // _S


def _softmax_desc(v):
    e = jnp.exp(v - v[:, :1])
    return e / jnp.sum(e, axis=1, keepdims=True)


def _write_out(val_ref, idx_ref, v, i):
    val_ref[...] = jnp.concatenate(
        [_softmax_desc(v), jnp.zeros((_BX, _LANES - _K), jnp.float32)], axis=1)
    idx_ref[...] = jnp.concatenate(
        [i, jnp.zeros((_BX, _LANES - _K), jnp.int32)], axis=1)


def _compute_sim(x_ref, yt_ref, bias_ref):
    sim = jax.lax.dot_general(
        x_ref[...], yt_ref[...], (((1,), (0,)), ((), ())),
        preferred_element_type=jnp.float32,
    ) / _TAU
    return sim + bias_ref[...]


def _body(x_ref, yt_ref, bias_ref, val_ref, idx_ref):
    sim = _compute_sim(x_ref, yt_ref, bias_ref)

    ninf = jnp.float32(-jnp.inf)
    m1 = jnp.full((_BX, _S), ninf)
    m2 = jnp.full((_BX, _S), ninf)
    a1 = jnp.zeros((_BX, _S), jnp.int32)
    a2 = jnp.zeros((_BX, _S), jnp.int32)
    for j in range(_NCH):
        v = sim[:, j * _S:(j + 1) * _S]
        jb = jnp.full((_BX, _S), j, jnp.int32)
        gt1 = v > m1
        gt2 = v > m2
        m2 = jnp.where(gt1, m1, jnp.where(gt2, v, m2))
        a2 = jnp.where(gt1, a1, jnp.where(gt2, jb, a2))
        m1 = jnp.where(gt1, v, m1)
        a1 = jnp.where(gt1, jb, a1)

    seg = jax.lax.broadcasted_iota(jnp.int32, (_BX, _S), 1)
    vals = []
    idxs = []
    for _ in range(_K):
        m = jnp.max(m1, axis=1, keepdims=True)
        s = jnp.min(jnp.where(m1 == m, seg, _S), axis=1, keepdims=True)
        oneh = seg == s
        c = jnp.sum(jnp.where(oneh, a1, 0), axis=1, keepdims=True)
        vals.append(m)
        idxs.append(c * _S + s)
        m1 = jnp.where(oneh, m2, m1)
        a1 = jnp.where(oneh, a2, a1)
        m2 = jnp.where(oneh, ninf, m2)

    _write_out(val_ref, idx_ref,
               jnp.concatenate(vals, axis=1),
               jnp.concatenate(idxs, axis=1))

    v15 = vals[-1]
    cnt = jnp.sum((sim > v15).astype(jnp.int32), axis=1, keepdims=True)
    redo = jnp.any(cnt > _K - 1)

    @pl.when(redo)
    def _redo():
        sm = _compute_sim(x_ref, yt_ref, bias_ref)
        colidx = jax.lax.broadcasted_iota(jnp.int32, (_BX, _NYP), 1)
        rvals = []
        ridxs = []
        for _ in range(_K):
            mm = jnp.max(sm, axis=1, keepdims=True)
            am = jnp.min(jnp.where(sm == mm, colidx, _NYP),
                         axis=1, keepdims=True)
            rvals.append(mm)
            ridxs.append(am)
            sm = jnp.where(colidx == am, ninf, sm)
        _write_out(val_ref, idx_ref,
                   jnp.concatenate(rvals, axis=1),
                   jnp.concatenate(ridxs, axis=1))


def kernel(feat_x, feat_y):
    yt = jnp.pad(feat_y.T, ((0, 0), (0, _NYP - _NY)))
    bias = jnp.where(jnp.arange(_NYP) < _NY, 0.0, -1e30)[None, :]
    bias = bias.astype(jnp.float32)

    sm_pad, idx_pad = pl.pallas_call(
        _body,
        grid=(_NX // _BX,),
        in_specs=[
            pl.BlockSpec((_BX, _C), lambda i: (i, 0)),
            pl.BlockSpec((_C, _NYP), lambda i: (0, 0)),
            pl.BlockSpec((1, _NYP), lambda i: (0, 0)),
        ],
        out_specs=[
            pl.BlockSpec((_BX, _LANES), lambda i: (i, 0)),
            pl.BlockSpec((_BX, _LANES), lambda i: (i, 0)),
        ],
        out_shape=[
            jax.ShapeDtypeStruct((_NX, _LANES), jnp.float32),
            jax.ShapeDtypeStruct((_NX, _LANES), jnp.int32),
        ],
    )(feat_x, yt, bias)

    row_idx = jnp.broadcast_to(
        jnp.arange(_NX, dtype=jnp.int64)[:, None], (_NX, _K))
    return (row_idx,
            idx_pad[:, :_K].astype(jnp.int64),
            sm_pad[:, :_K])


# speedup vs baseline: 7.8265x; 1.5778x over previous
import jax
import jax.numpy as jnp
from jax.experimental import pallas as pl

_TAU = 0.2
_K = 15
_NX = 1024
_NY = 100000
_C = 16
_LANES = 128
_NYP = 100352
_BX = 16
_S = 512
_NCH = _NYP // _S


def _softmax_desc(v):
    e = jnp.exp(v - v[:, :1])
    return e / jnp.sum(e, axis=1, keepdims=True)


def _write_out(val_ref, idx_ref, v, i):
    val_ref[...] = jnp.concatenate(
        [_softmax_desc(v), jnp.zeros((_BX, _LANES - _K), jnp.float32)], axis=1)
    idx_ref[...] = jnp.concatenate(
        [i, jnp.zeros((_BX, _LANES - _K), jnp.int32)], axis=1)


def _compute_sim(x_ref, yt_ref, bias_ref):
    sim = jax.lax.dot_general(
        x_ref[...], yt_ref[...], (((1,), (0,)), ((), ())),
        preferred_element_type=jnp.float32,
    ) / _TAU
    return sim + bias_ref[...]


def _body(x_ref, yt_ref, bias_ref, val_ref, idx_ref):
    sim = _compute_sim(x_ref, yt_ref, bias_ref)

    ninf = jnp.float32(-jnp.inf)
    m1 = jnp.full((_BX, _S), ninf)
    m2 = jnp.full((_BX, _S), ninf)
    a1 = jnp.zeros((_BX, _S), jnp.int32)
    a2 = jnp.zeros((_BX, _S), jnp.int32)
    for j in range(_NCH):
        v = sim[:, j * _S:(j + 1) * _S]
        jb = jnp.full((_BX, _S), j, jnp.int32)
        gt1 = v > m1
        gt2 = v > m2
        m2 = jnp.where(gt1, m1, jnp.where(gt2, v, m2))
        a2 = jnp.where(gt1, a1, jnp.where(gt2, jb, a2))
        m1 = jnp.where(gt1, v, m1)
        a1 = jnp.where(gt1, jb, a1)

    seg = jax.lax.broadcasted_iota(jnp.int32, (_BX, _S), 1)
    vals = []
    idxs = []
    for _ in range(_K):
        m = jnp.max(m1, axis=1, keepdims=True)
        s = jnp.min(jnp.where(m1 == m, seg, _S), axis=1, keepdims=True)
        oneh = seg == s
        c = jnp.sum(jnp.where(oneh, a1, 0), axis=1, keepdims=True)
        vals.append(m)
        idxs.append(c * _S + s)
        m1 = jnp.where(oneh, m2, m1)
        a1 = jnp.where(oneh, a2, a1)
        m2 = jnp.where(oneh, ninf, m2)

    _write_out(val_ref, idx_ref,
               jnp.concatenate(vals, axis=1),
               jnp.concatenate(idxs, axis=1))

    v15 = vals[-1]
    cnt = jnp.sum((sim > v15).astype(jnp.int32), axis=1, keepdims=True)
    redo = jnp.any(cnt > _K - 1)

    @pl.when(redo)
    def _redo():
        sm = _compute_sim(x_ref, yt_ref, bias_ref)
        colidx = jax.lax.broadcasted_iota(jnp.int32, (_BX, _NYP), 1)
        rvals = []
        ridxs = []
        for _ in range(_K):
            mm = jnp.max(sm, axis=1, keepdims=True)
            am = jnp.min(jnp.where(sm == mm, colidx, _NYP),
                         axis=1, keepdims=True)
            rvals.append(mm)
            ridxs.append(am)
            sm = jnp.where(colidx == am, ninf, sm)
        _write_out(val_ref, idx_ref,
                   jnp.concatenate(rvals, axis=1),
                   jnp.concatenate(ridxs, axis=1))


def kernel(feat_x, feat_y):
    yt = jnp.pad(feat_y.T, ((0, 0), (0, _NYP - _NY)))
    bias = jnp.where(jnp.arange(_NYP) < _NY, 0.0, -1e30)[None, :]
    bias = bias.astype(jnp.float32)

    sm_pad, idx_pad = pl.pallas_call(
        _body,
        grid=(_NX // _BX,),
        in_specs=[
            pl.BlockSpec((_BX, _C), lambda i: (i, 0)),
            pl.BlockSpec((_C, _NYP), lambda i: (0, 0)),
            pl.BlockSpec((1, _NYP), lambda i: (0, 0)),
        ],
        out_specs=[
            pl.BlockSpec((_BX, _LANES), lambda i: (i, 0)),
            pl.BlockSpec((_BX, _LANES), lambda i: (i, 0)),
        ],
        out_shape=[
            jax.ShapeDtypeStruct((_NX, _LANES), jnp.float32),
            jax.ShapeDtypeStruct((_NX, _LANES), jnp.int32),
        ],
    )(feat_x, yt, bias)

    row_idx = jnp.broadcast_to(
        jnp.arange(_NX, dtype=jnp.int64)[:, None], (_NX, _K))
    return (row_idx,
            idx_pad[:, :_K].astype(jnp.int64),
            sm_pad[:, :_K])


# speedup vs baseline: 9.1484x; 1.1689x over previous
import jax
import jax.numpy as jnp
from jax.experimental import pallas as pl

_TAU = 0.2
_K = 15
_NX = 1024
_NY = 100000
_C = 16
_LANES = 128
_NYP = 100352
_BX = 32
_S = 512
_NCH = _NYP // _S


def _softmax_desc(v):
    e = jnp.exp(v - v[:, :1])
    return e / jnp.sum(e, axis=1, keepdims=True)


def _write_out(val_ref, idx_ref, v, i):
    val_ref[...] = jnp.concatenate(
        [_softmax_desc(v), jnp.zeros((_BX, _LANES - _K), jnp.float32)], axis=1)
    idx_ref[...] = jnp.concatenate(
        [i, jnp.zeros((_BX, _LANES - _K), jnp.int32)], axis=1)


def _compute_sim(x_ref, yt_ref, bias_ref):
    sim = jax.lax.dot_general(
        x_ref[...], yt_ref[...], (((1,), (0,)), ((), ())),
        preferred_element_type=jnp.float32,
    ) / _TAU
    return sim + bias_ref[...]


def _body(x_ref, yt_ref, bias_ref, val_ref, idx_ref):
    sim = _compute_sim(x_ref, yt_ref, bias_ref)

    ninf = jnp.float32(-jnp.inf)
    m1 = jnp.full((_BX, _S), ninf)
    m2 = jnp.full((_BX, _S), ninf)
    a1 = jnp.zeros((_BX, _S), jnp.int32)
    a2 = jnp.zeros((_BX, _S), jnp.int32)
    for j in range(_NCH):
        v = sim[:, j * _S:(j + 1) * _S]
        jb = jnp.full((_BX, _S), j, jnp.int32)
        gt1 = v > m1
        gt2 = v > m2
        m2 = jnp.where(gt1, m1, jnp.where(gt2, v, m2))
        a2 = jnp.where(gt1, a1, jnp.where(gt2, jb, a2))
        m1 = jnp.where(gt1, v, m1)
        a1 = jnp.where(gt1, jb, a1)

    seg = jax.lax.broadcasted_iota(jnp.int32, (_BX, _S), 1)
    vals = []
    idxs = []
    for _ in range(_K):
        m = jnp.max(m1, axis=1, keepdims=True)
        s = jnp.min(jnp.where(m1 == m, seg, _S), axis=1, keepdims=True)
        oneh = seg == s
        c = jnp.sum(jnp.where(oneh, a1, 0), axis=1, keepdims=True)
        vals.append(m)
        idxs.append(c * _S + s)
        m1 = jnp.where(oneh, m2, m1)
        a1 = jnp.where(oneh, a2, a1)
        m2 = jnp.where(oneh, ninf, m2)

    _write_out(val_ref, idx_ref,
               jnp.concatenate(vals, axis=1),
               jnp.concatenate(idxs, axis=1))

    v15 = vals[-1]
    cnt = jnp.sum((sim > v15).astype(jnp.int32), axis=1, keepdims=True)
    redo = jnp.any(cnt > _K - 1)

    @pl.when(redo)
    def _redo():
        sm = _compute_sim(x_ref, yt_ref, bias_ref)
        colidx = jax.lax.broadcasted_iota(jnp.int32, (_BX, _NYP), 1)
        rvals = []
        ridxs = []
        for _ in range(_K):
            mm = jnp.max(sm, axis=1, keepdims=True)
            am = jnp.min(jnp.where(sm == mm, colidx, _NYP),
                         axis=1, keepdims=True)
            rvals.append(mm)
            ridxs.append(am)
            sm = jnp.where(colidx == am, ninf, sm)
        _write_out(val_ref, idx_ref,
                   jnp.concatenate(rvals, axis=1),
                   jnp.concatenate(ridxs, axis=1))


def kernel(feat_x, feat_y):
    yt = jnp.pad(feat_y.T, ((0, 0), (0, _NYP - _NY)))
    bias = jnp.where(jnp.arange(_NYP) < _NY, 0.0, -1e30)[None, :]
    bias = bias.astype(jnp.float32)

    sm_pad, idx_pad = pl.pallas_call(
        _body,
        grid=(_NX // _BX,),
        in_specs=[
            pl.BlockSpec((_BX, _C), lambda i: (i, 0)),
            pl.BlockSpec((_C, _NYP), lambda i: (0, 0)),
            pl.BlockSpec((1, _NYP), lambda i: (0, 0)),
        ],
        out_specs=[
            pl.BlockSpec((_BX, _LANES), lambda i: (i, 0)),
            pl.BlockSpec((_BX, _LANES), lambda i: (i, 0)),
        ],
        out_shape=[
            jax.ShapeDtypeStruct((_NX, _LANES), jnp.float32),
            jax.ShapeDtypeStruct((_NX, _LANES), jnp.int32),
        ],
    )(feat_x, yt, bias)

    row_idx = jnp.broadcast_to(
        jnp.arange(_NX, dtype=jnp.int64)[:, None], (_NX, _K))
    return (row_idx,
            idx_pad[:, :_K].astype(jnp.int64),
            sm_pad[:, :_K])
